# baseline (speedup 1.0000x reference)
import functools

import jax
import jax.numpy as jnp
from jax import lax
from jax.experimental import pallas as pl
from jax.experimental.pallas import tpu as pltpu
from jax.experimental.pallas import tpu_sc as plsc

B = 128
C = 5
HW = 256
PLANE = C * HW
NSUB = 16
SPB = B // NSUB
SLAB = SPB * PLANE
L = 16
INV_S = 1.0 / 32.0
NOOBJ = 0.5
COORD = 5.0


def _sqrt16(a):
    i = lax.bitcast_convert_type(a, jnp.int32)
    i = jnp.int32(0x5F3759DF) - lax.shift_right_logical(i, 1)
    y = lax.bitcast_convert_type(i, jnp.float32)
    for _ in range(4):
        y = y * (1.5 - 0.5 * a * y * y)
    return a * y


def _sc_loss(preds_flat, x, y, w, h, label):
    mesh = plsc.VectorSubcoreMesh(
        core_axis_name="c", subcore_axis_name="s", num_cores=1)

    @functools.partial(
        pl.kernel,
        mesh=mesh,
        compiler_params=pltpu.CompilerParams(needs_layout_passes=False),
        out_type=jax.ShapeDtypeStruct((NSUB, L), jnp.float32),
        scratch_types=[
            pltpu.VMEM((SLAB,), jnp.float32),
            pltpu.VMEM((B,), jnp.float32),
            pltpu.VMEM((B,), jnp.float32),
            pltpu.VMEM((B,), jnp.float32),
            pltpu.VMEM((B,), jnp.float32),
            pltpu.VMEM((B,), jnp.float32),
            pltpu.VMEM((L,), jnp.float32),
        ],
    )
    def k(preds_hbm, x_hbm, y_hbm, w_hbm, h_hbm, label_hbm, out_hbm,
          buf, xv, yv, wv, hv, lv, pvec):
        sid = lax.axis_index("s")
        base = sid * SPB

        pltpu.sync_copy(preds_hbm.at[pl.ds(sid * SLAB, SLAB)], buf)
        pltpu.sync_copy(x_hbm, xv)
        pltpu.sync_copy(y_hbm, yv)
        pltpu.sync_copy(w_hbm, wv)
        pltpu.sync_copy(h_hbm, hv)
        pltpu.sync_copy(label_hbm, lv)

        acc = jnp.zeros((L,), jnp.float32)
        for s in range(SPB):
            for i in range(HW // L):
                v = buf[pl.ds(s * PLANE + i * L, L)]
                acc = acc + v * v

        lane = lax.iota(jnp.int32, L)
        sv = jnp.minimum(lane, SPB - 1)
        mask = lane < SPB

        bidx = base + sv
        xg = plsc.load_gather(xv, [bidx])
        yg = plsc.load_gather(yv, [bidx])
        wg = plsc.load_gather(wv, [bidx])
        hg = plsc.load_gather(hv, [bidx])
        lg = plsc.load_gather(lv, [bidx])

        tx = (lax.rem(xg, 32.0) * INV_S - 0.5) * lg
        ty = (lax.rem(yg, 32.0) * INV_S - 0.5) * lg
        xi = (xg * INV_S).astype(jnp.int32)
        yi = (yg * INV_S).astype(jnp.int32)
        cell = xi * 16 + yi
        stw = _sqrt16(wg * INV_S)
        sth = _sqrt16(hg * INV_S)

        cbase = sv * PLANE + cell
        c0 = plsc.load_gather(buf, [cbase])
        c1 = plsc.load_gather(buf, [cbase + HW])
        c2 = plsc.load_gather(buf, [cbase + 2 * HW])
        c3 = plsc.load_gather(buf, [cbase + 3 * HW])
        c4 = plsc.load_gather(buf, [cbase + 4 * HW])

        dw = stw - _sqrt16(c3)
        dh = sth - _sqrt16(c4)
        term = ((1.0 - c0) * (1.0 - c0) - NOOBJ * c0 * c0
                + COORD * ((tx - c1) * (tx - c1)
                           + (ty - c2) * (ty - c2)
                           + dw * dw + dh * dh))
        partial = NOOBJ * acc + jnp.where(mask, term, 0.0)

        pvec[...] = partial
        pltpu.sync_copy(pvec, out_hbm.at[sid])

    return k(preds_flat, x, y, w, h, label)


def _tc_reduce(part):
    def body(p_ref, o_ref):
        o_ref[0, 0] = jnp.sum(p_ref[...])

    return pl.pallas_call(
        body,
        out_shape=jax.ShapeDtypeStruct((1, 1), jnp.float32),
        out_specs=pl.BlockSpec(memory_space=pltpu.SMEM),
    )(part)


def kernel(preds, x, y, w, h, label):
    part = _sc_loss(preds.reshape(-1), x, y, w, h, label)
    return _tc_reduce(part)[0, 0]


# speedup vs baseline: 1.0527x; 1.0527x over previous
import functools

import jax
import jax.numpy as jnp
from jax import lax
from jax.experimental import pallas as pl
from jax.experimental.pallas import tpu as pltpu
from jax.experimental.pallas import tpu_sc as plsc

B = 128
C = 5
HW = 256
PLANE = C * HW
NSUB = 16
SPB = B // NSUB
SLAB = SPB * PLANE
L = 16
INV_S = 1.0 / 32.0
NOOBJ = 0.5
COORD = 5.0


def _sqrt16(a):
    i = lax.bitcast_convert_type(a, jnp.int32)
    i = jnp.int32(0x5F3759DF) - lax.shift_right_logical(i, 1)
    y = lax.bitcast_convert_type(i, jnp.float32)
    for _ in range(4):
        y = y * (1.5 - 0.5 * a * y * y)
    return a * y


def _sc_loss(preds_flat, x, y, w, h, label):
    mesh = plsc.VectorSubcoreMesh(
        core_axis_name="c", subcore_axis_name="s", num_cores=1)

    @functools.partial(
        pl.kernel,
        mesh=mesh,
        compiler_params=pltpu.CompilerParams(needs_layout_passes=False),
        out_type=jax.ShapeDtypeStruct((L,), jnp.float32),
        scratch_types=[
            pltpu.VMEM((SLAB,), jnp.float32),
            pltpu.VMEM((B,), jnp.float32),
            pltpu.VMEM((B,), jnp.float32),
            pltpu.VMEM((B,), jnp.float32),
            pltpu.VMEM((B,), jnp.float32),
            pltpu.VMEM((B,), jnp.float32),
            pltpu.VMEM((L,), jnp.float32),
            pltpu.VMEM_SHARED((L,), jnp.float32),
            pltpu.VMEM((L,), jnp.float32),
        ],
    )
    def k(preds_hbm, x_hbm, y_hbm, w_hbm, h_hbm, label_hbm, out_hbm,
          buf, xv, yv, wv, hv, lv, pvec, sacc, outv):
        sid = lax.axis_index("s")
        base = sid * SPB

        @pl.when(sid == 0)
        def _():
            outv[...] = jnp.zeros((L,), jnp.float32)
            pltpu.sync_copy(outv, sacc)

        pltpu.sync_copy(preds_hbm.at[pl.ds(sid * SLAB, SLAB)], buf)
        pltpu.sync_copy(x_hbm, xv)
        pltpu.sync_copy(y_hbm, yv)
        pltpu.sync_copy(w_hbm, wv)
        pltpu.sync_copy(h_hbm, hv)
        pltpu.sync_copy(label_hbm, lv)

        a0 = jnp.zeros((L,), jnp.float32)
        a1 = jnp.zeros((L,), jnp.float32)
        a2 = jnp.zeros((L,), jnp.float32)
        a3 = jnp.zeros((L,), jnp.float32)
        for s in range(SPB):
            for i in range(0, HW // L, 4):
                v0 = buf[pl.ds(s * PLANE + i * L, L)]
                v1 = buf[pl.ds(s * PLANE + (i + 1) * L, L)]
                v2 = buf[pl.ds(s * PLANE + (i + 2) * L, L)]
                v3 = buf[pl.ds(s * PLANE + (i + 3) * L, L)]
                a0 = a0 + v0 * v0
                a1 = a1 + v1 * v1
                a2 = a2 + v2 * v2
                a3 = a3 + v3 * v3
        acc = (a0 + a1) + (a2 + a3)

        lane = lax.iota(jnp.int32, L)
        sv = jnp.minimum(lane, SPB - 1)
        mask = lane < SPB

        bidx = base + sv
        xg = plsc.load_gather(xv, [bidx])
        yg = plsc.load_gather(yv, [bidx])
        wg = plsc.load_gather(wv, [bidx])
        hg = plsc.load_gather(hv, [bidx])
        lg = plsc.load_gather(lv, [bidx])

        tx = (lax.rem(xg, 32.0) * INV_S - 0.5) * lg
        ty = (lax.rem(yg, 32.0) * INV_S - 0.5) * lg
        xi = (xg * INV_S).astype(jnp.int32)
        yi = (yg * INV_S).astype(jnp.int32)
        cell = xi * 16 + yi
        stw = _sqrt16(wg * INV_S)
        sth = _sqrt16(hg * INV_S)

        cbase = sv * PLANE + cell
        c0 = plsc.load_gather(buf, [cbase])
        c1 = plsc.load_gather(buf, [cbase + HW])
        c2 = plsc.load_gather(buf, [cbase + 2 * HW])
        c3 = plsc.load_gather(buf, [cbase + 3 * HW])
        c4 = plsc.load_gather(buf, [cbase + 4 * HW])

        dw = stw - _sqrt16(c3)
        dh = sth - _sqrt16(c4)
        term = ((1.0 - c0) * (1.0 - c0) - NOOBJ * c0 * c0
                + COORD * ((tx - c1) * (tx - c1)
                           + (ty - c2) * (ty - c2)
                           + dw * dw + dh * dh))
        partial = NOOBJ * acc + jnp.where(mask, term, 0.0)

        plsc.subcore_barrier()
        pvec[...] = partial
        pltpu.sync_copy(pvec, sacc.at[jnp.arange(L, dtype=jnp.int32)], add=True)
        plsc.subcore_barrier()

        @pl.when(sid == 0)
        def _():
            pltpu.sync_copy(sacc, pvec)
            outv[...] = jnp.full((L,), jnp.sum(pvec[...]), jnp.float32)
            pltpu.sync_copy(outv, out_hbm)

    return k(preds_flat, x, y, w, h, label)


def kernel(preds, x, y, w, h, label):
    out = _sc_loss(preds.reshape(-1), x, y, w, h, label)
    return out[0]


# speedup vs baseline: 1.1867x; 1.1272x over previous
import functools

import jax
import jax.numpy as jnp
from jax import lax
from jax.experimental import pallas as pl
from jax.experimental.pallas import tpu as pltpu
from jax.experimental.pallas import tpu_sc as plsc

B = 128
C = 5
HW = 256
PLANE = C * HW
NSUB = 16
SPB = B // NSUB
SLAB = SPB * PLANE
L = 16
INV_S = 1.0 / 32.0
NOOBJ = 0.5
COORD = 5.0


def _sqrt16(a):
    i = lax.bitcast_convert_type(a, jnp.int32)
    i = jnp.int32(0x5F3759DF) - lax.shift_right_logical(i, 1)
    y = lax.bitcast_convert_type(i, jnp.float32)
    for _ in range(4):
        y = y * (1.5 - 0.5 * a * y * y)
    return a * y


def _sc_loss(preds_flat, x, y, w, h, label):
    mesh = plsc.VectorSubcoreMesh(
        core_axis_name="c", subcore_axis_name="s", num_cores=1)

    @functools.partial(
        pl.kernel,
        mesh=mesh,
        compiler_params=pltpu.CompilerParams(needs_layout_passes=False),
        out_type=jax.ShapeDtypeStruct((L,), jnp.float32),
        scratch_types=[
            pltpu.VMEM((SLAB,), jnp.float32),
            pltpu.VMEM((SPB,), jnp.float32),
            pltpu.VMEM((SPB,), jnp.float32),
            pltpu.VMEM((SPB,), jnp.float32),
            pltpu.VMEM((SPB,), jnp.float32),
            pltpu.VMEM((SPB,), jnp.float32),
            pltpu.VMEM((L,), jnp.float32),
            pltpu.VMEM_SHARED((L,), jnp.float32),
            pltpu.VMEM((L,), jnp.float32),
            pltpu.SemaphoreType.DMA,
        ],
    )
    def k(preds_hbm, x_hbm, y_hbm, w_hbm, h_hbm, label_hbm, out_hbm,
          buf, xv, yv, wv, hv, lv, pvec, sacc, outv, sem):
        sid = lax.axis_index("s")
        base = sid * SPB

        @pl.when(sid == 0)
        def _():
            outv[...] = jnp.zeros((L,), jnp.float32)
            pltpu.sync_copy(outv, sacc)

        cps = [
            pltpu.async_copy(preds_hbm.at[pl.ds(sid * SLAB, SLAB)], buf, sem),
            pltpu.async_copy(x_hbm.at[pl.ds(base, SPB)], xv, sem),
            pltpu.async_copy(y_hbm.at[pl.ds(base, SPB)], yv, sem),
            pltpu.async_copy(w_hbm.at[pl.ds(base, SPB)], wv, sem),
            pltpu.async_copy(h_hbm.at[pl.ds(base, SPB)], hv, sem),
            pltpu.async_copy(label_hbm.at[pl.ds(base, SPB)], lv, sem),
        ]
        for cp in cps:
            cp.wait()

        a0 = jnp.zeros((L,), jnp.float32)
        a1 = jnp.zeros((L,), jnp.float32)
        a2 = jnp.zeros((L,), jnp.float32)
        a3 = jnp.zeros((L,), jnp.float32)
        for s in range(SPB):
            for i in range(0, HW // L, 4):
                v0 = buf[pl.ds(s * PLANE + i * L, L)]
                v1 = buf[pl.ds(s * PLANE + (i + 1) * L, L)]
                v2 = buf[pl.ds(s * PLANE + (i + 2) * L, L)]
                v3 = buf[pl.ds(s * PLANE + (i + 3) * L, L)]
                a0 = a0 + v0 * v0
                a1 = a1 + v1 * v1
                a2 = a2 + v2 * v2
                a3 = a3 + v3 * v3
        acc = (a0 + a1) + (a2 + a3)

        lane = lax.iota(jnp.int32, L)
        sv = jnp.minimum(lane, SPB - 1)
        mask = lane < SPB

        xg = plsc.load_gather(xv, [sv])
        yg = plsc.load_gather(yv, [sv])
        wg = plsc.load_gather(wv, [sv])
        hg = plsc.load_gather(hv, [sv])
        lg = plsc.load_gather(lv, [sv])

        tx = (lax.rem(xg, 32.0) * INV_S - 0.5) * lg
        ty = (lax.rem(yg, 32.0) * INV_S - 0.5) * lg
        xi = (xg * INV_S).astype(jnp.int32)
        yi = (yg * INV_S).astype(jnp.int32)
        cell = xi * 16 + yi
        stw = _sqrt16(wg * INV_S)
        sth = _sqrt16(hg * INV_S)

        cbase = sv * PLANE + cell
        c0 = plsc.load_gather(buf, [cbase])
        c1 = plsc.load_gather(buf, [cbase + HW])
        c2 = plsc.load_gather(buf, [cbase + 2 * HW])
        c3 = plsc.load_gather(buf, [cbase + 3 * HW])
        c4 = plsc.load_gather(buf, [cbase + 4 * HW])

        dw = stw - _sqrt16(c3)
        dh = sth - _sqrt16(c4)
        term = ((1.0 - c0) * (1.0 - c0) - NOOBJ * c0 * c0
                + COORD * ((tx - c1) * (tx - c1)
                           + (ty - c2) * (ty - c2)
                           + dw * dw + dh * dh))
        partial = NOOBJ * acc + jnp.where(mask, term, 0.0)

        plsc.subcore_barrier()
        pvec[...] = partial
        pltpu.sync_copy(pvec, sacc.at[jnp.arange(L, dtype=jnp.int32)], add=True)
        plsc.subcore_barrier()

        @pl.when(sid == 0)
        def _():
            pltpu.sync_copy(sacc, pvec)
            outv[...] = jnp.full((L,), jnp.sum(pvec[...]), jnp.float32)
            pltpu.sync_copy(outv, out_hbm)

    return k(preds_flat, x, y, w, h, label)


def kernel(preds, x, y, w, h, label):
    out = _sc_loss(preds.reshape(-1), x, y, w, h, label)
    return out[0]
